# baseline (speedup 1.0000x reference)
import functools

import jax
import jax.numpy as jnp
from jax import lax
from jax.experimental import pallas as pl
from jax.experimental.pallas import tpu as pltpu
from jax.experimental.pallas import tpu_sc as plsc

N = 10000
E = 320000
D_IN = 128
D_HID = 64
D_OUT = 32

NC = 2
NS = 16
NW = NC * NS
EPW = E // NW
C = 80
MCH = EPW // C
RPT = 640
NP = RPT * NS
NF = N // 2
NFP = NP // 2

_mesh = plsc.VectorSubcoreMesh(core_axis_name="c", subcore_axis_name="s")


def _deg_body(dst_hbm, out_hbm, didx, ones_v, zrow_v, dbuf, fbuf, acc):
    c = lax.axis_index("c")
    s = lax.axis_index("s")
    wid = s * NC + c
    pltpu.sync_copy(dst_hbm.at[wid], didx)
    one16 = jnp.full((16,), 1.0, dtype=jnp.float32)
    zero16 = jnp.zeros((16,), dtype=jnp.float32)
    for j in range(C // 16):
        ones_v[pl.ds(j * 16, 16)] = one16

    def _z(i, carry):
        zrow_v[pl.ds(i * 16, 16)] = zero16
        return carry

    lax.fori_loop(0, RPT // 16, _z, 0)
    base = s * RPT
    pltpu.sync_copy(zrow_v, acc.at[pl.ds(base, RPT)])
    plsc.subcore_barrier()

    def _chunk(i, carry):
        pltpu.sync_copy(ones_v, acc.at[didx.at[i]], add=True)
        return carry

    lax.fori_loop(0, MCH, _chunk, 0)
    plsc.subcore_barrier()
    pltpu.sync_copy(acc.at[pl.ds(base, RPT)], dbuf)

    def _b(i, carry):
        d16 = dbuf[pl.ds(i * 16, 16)]
        for k in range(16):
            vk = jnp.broadcast_to(d16[k], (16,))
            r = 8 * i + k // 2
            off = (k % 2) * 64
            for j in range(4):
                fbuf[r, pl.ds(off + j * 16, 16)] = vk
        return carry

    lax.fori_loop(0, RPT // 16, _b, 0)
    pltpu.sync_copy(fbuf, out_hbm.at[c, pl.ds(s * (RPT // 2), RPT // 2)])


_deg_call = pl.kernel(
    _deg_body,
    out_type=jax.ShapeDtypeStruct((NC, NFP, 2 * D_HID), jnp.float32),
    mesh=_mesh,
    scratch_types=[
        pltpu.VMEM((MCH, C), jnp.int32),
        pltpu.VMEM((C,), jnp.float32),
        pltpu.VMEM((RPT,), jnp.float32),
        pltpu.VMEM((RPT,), jnp.float32),
        pltpu.VMEM((RPT // 2, 2 * D_HID), jnp.float32),
        pltpu.VMEM_SHARED((NP,), jnp.float32),
    ],
    compiler_params=pltpu.CompilerParams(use_tc_tiling_on_sc=False),
)


G = 6
NIT = (MCH + 2 * G - 1) // (2 * G)


def _prop_body(src_hbm, dst_hbm, z_hbm, out_hbm, sidx, didx, *rest):
    A = list(rest[:G])
    B = list(rest[G:2 * G])
    tmp, acc, gsa, ssa, gsb, ssb = rest[2 * G:]
    c = lax.axis_index("c")
    s = lax.axis_index("s")
    wid = s * NC + c
    pltpu.sync_copy(src_hbm.at[wid], sidx)
    pltpu.sync_copy(dst_hbm.at[wid], didx)

    def g_start(k, buf, sem):
        pltpu.async_copy(z_hbm.at[sidx.at[k]], buf, sem)

    def g_wait(k, buf, sem):
        pltpu.make_async_copy(z_hbm.at[sidx.at[k]], buf, sem).wait()

    def s_start(k, buf, sem):
        pltpu.async_copy(buf, acc.at[didx.at[k]], sem, add=True)

    def s_wait(k, buf, sem):
        pltpu.make_async_copy(buf, acc.at[didx.at[k]], sem).wait()

    for j in range(G):
        g_start(j, A[j], gsa.at[j])

    zero16 = jnp.zeros((16,), dtype=jnp.float32)

    def _z(i, carry):
        for j in range(D_HID // 16):
            tmp[i, pl.ds(j * 16, 16)] = zero16
        return carry

    lax.fori_loop(0, 128, _z, 0)
    base = s * RPT
    for j in range(RPT // 128):
        pltpu.sync_copy(tmp, acc.at[pl.ds(base + j * 128, 128)])
    plsc.subcore_barrier()

    def _iter(it, carry):
        i = it * 2 * G
        for j in range(G):
            def _a(j=j):
                g_wait(i + j, A[j], gsa.at[j])
                s_start(i + j, A[j], ssa.at[j])
            pl.when(i + j < MCH)(_a)
        for j in range(G):
            def _bd(j=j):
                s_wait(i - G + j, B[j], ssb.at[j])
            pl.when(it > 0)(_bd)
        for j in range(G):
            def _bg(j=j):
                g_start(i + G + j, B[j], gsb.at[j])
            pl.when(i + G + j < MCH)(_bg)
        for j in range(G):
            def _bw(j=j):
                g_wait(i + G + j, B[j], gsb.at[j])
            pl.when(i + G + j < MCH)(_bw)
        for j in range(G):
            def _ad(j=j):
                s_wait(i + j, A[j], ssa.at[j])
            pl.when(i + j < MCH)(_ad)
        for j in range(G):
            def _bs(j=j):
                s_start(i + G + j, B[j], ssb.at[j])
            pl.when(i + G + j < MCH)(_bs)
        for j in range(G):
            def _ag(j=j):
                g_start(i + 2 * G + j, A[j], gsa.at[j])
            pl.when(i + 2 * G + j < MCH)(_ag)
        return carry

    lax.fori_loop(0, NIT, _iter, 0)
    for j in range(G):
        k = (NIT - 1) * 2 * G + G + j
        if k < MCH:
            s_wait(k, B[j], ssb.at[j])
    plsc.subcore_barrier()
    for j in range(RPT // 128):
        sl = pl.ds(base + j * 128, 128)
        pltpu.sync_copy(acc.at[sl], out_hbm.at[c, sl])


_prop_call = pl.kernel(
    _prop_body,
    out_type=jax.ShapeDtypeStruct((NC, NP, D_HID), jnp.float32),
    mesh=_mesh,
    scratch_types=[
        pltpu.VMEM((MCH, C), jnp.int32),
        pltpu.VMEM((MCH, C), jnp.int32),
    ] + [pltpu.VMEM((C, D_HID), jnp.float32)] * (2 * G) + [
        pltpu.VMEM((128, D_HID), jnp.float32),
        pltpu.VMEM_SHARED((NP, D_HID), jnp.float32),
        pltpu.SemaphoreType.DMA((G,)),
        pltpu.SemaphoreType.DMA((G,)),
        pltpu.SemaphoreType.DMA((G,)),
        pltpu.SemaphoreType.DMA((G,)),
    ],
    compiler_params=pltpu.CompilerParams(use_tc_tiling_on_sc=False),
)


def _tcb_body(xf_ref, w2_ref, degf_ref, z1f_ref, dinvf_ref):
    deg = degf_ref[0][:NF] + degf_ref[1][:NF] + 1.0
    dinv = lax.rsqrt(deg)
    y = jnp.dot(xf_ref[...], w2_ref[...], preferred_element_type=jnp.float32)
    z1f_ref[...] = y * dinv
    dinvf_ref[...] = dinv


_tcb_call = pl.pallas_call(
    _tcb_body,
    out_shape=[
        jax.ShapeDtypeStruct((NF, 2 * D_HID), jnp.float32),
        jax.ShapeDtypeStruct((NF, 2 * D_HID), jnp.float32),
    ],
)


def _tcc_body(t1f_ref, z1f_ref, dinvf_ref, b1_ref, wc2_ref, z2f_ref):
    t = t1f_ref[0][:NF] + t1f_ref[1][:NF] + z1f_ref[...]
    dinv = dinvf_ref[...]
    h = jnp.maximum(dinv * t + b1_ref[...], 0.0)
    y2 = jnp.dot(h, wc2_ref[...], preferred_element_type=jnp.float32)
    z2f_ref[...] = y2 * dinv


_tcc_call = pl.pallas_call(
    _tcc_body,
    out_shape=jax.ShapeDtypeStruct((NF, 2 * D_HID), jnp.float32),
)


def _tcd_body(t2f_ref, z2f_ref, dinvf_ref, bc2_ref, of_ref):
    t = t2f_ref[0][:NF] + t2f_ref[1][:NF] + z2f_ref[...]
    of_ref[...] = dinvf_ref[...] * t + bc2_ref[...]


_tcd_call = pl.pallas_call(
    _tcd_body,
    out_shape=jax.ShapeDtypeStruct((NF, 2 * D_HID), jnp.float32),
)


def kernel(x, edge_index, W1, b1, Wmu, bmu, Wsig, bsig):
    f32 = jnp.float32
    src = edge_index[0].astype(jnp.int32).reshape(NW, MCH, C)
    dst = edge_index[1].astype(jnp.int32).reshape(NW, MCH, C)
    degf = _deg_call(dst)
    W2 = jnp.zeros((2 * D_IN, 2 * D_HID), f32)
    W2 = W2.at[:D_IN, :D_HID].set(W1).at[D_IN:, D_HID:].set(W1)
    wcat = jnp.concatenate([Wmu, Wsig], axis=1)
    wc2 = jnp.zeros((2 * D_HID, 2 * D_HID), f32)
    wc2 = wc2.at[:D_HID, :D_HID].set(wcat).at[D_HID:, D_HID:].set(wcat)
    b1_2 = jnp.concatenate([b1, b1])[None, :]
    bc2 = jnp.concatenate([bmu, bsig, bmu, bsig])[None, :]
    xf = x.reshape(NF, 2 * D_IN)
    z1f, dinvf = _tcb_call(xf, W2, degf)
    t1 = _prop_call(src, dst, z1f.reshape(N, D_HID))
    z2f = _tcc_call(t1.reshape(NC, NFP, 2 * D_HID), z1f, dinvf, b1_2, wc2)
    t2 = _prop_call(src, dst, z2f.reshape(N, D_HID))
    of = _tcd_call(t2.reshape(NC, NFP, 2 * D_HID), z2f, dinvf, bc2)
    o = of.reshape(N, D_HID)
    return o[:, :D_OUT], o[:, D_OUT:]


# speedup vs baseline: 1.0415x; 1.0415x over previous
import functools

import jax
import jax.numpy as jnp
from jax import lax
from jax.experimental import pallas as pl
from jax.experimental.pallas import tpu as pltpu
from jax.experimental.pallas import tpu_sc as plsc

N = 10000
E = 320000
D_IN = 128
D_HID = 64
D_OUT = 32

NC = 2
NS = 16
NW = NC * NS
EPW = E // NW
C = 80
MCH = EPW // C
RPT = 640
NP = RPT * NS
NF = N // 2
NFP = NP // 2

_mesh = plsc.VectorSubcoreMesh(core_axis_name="c", subcore_axis_name="s")


def _deg_body(dst_hbm, out_hbm, didx, ones_v, zrow_v, dbuf, fbuf, acc, dsem):
    c = lax.axis_index("c")
    s = lax.axis_index("s")
    wid = s * NC + c
    pltpu.sync_copy(dst_hbm.at[wid], didx)
    one16 = jnp.full((16,), 1.0, dtype=jnp.float32)
    zero16 = jnp.zeros((16,), dtype=jnp.float32)
    for j in range(C // 16):
        ones_v[pl.ds(j * 16, 16)] = one16

    def _z(i, carry):
        zrow_v[pl.ds(i * 16, 16)] = zero16
        return carry

    lax.fori_loop(0, RPT // 16, _z, 0)
    base = s * RPT
    pltpu.sync_copy(zrow_v, acc.at[pl.ds(base, RPT)])
    plsc.subcore_barrier()

    def _chunk(i, carry):
        pltpu.async_copy(ones_v, acc.at[didx.at[i]], dsem, add=True)
        return carry

    lax.fori_loop(0, MCH, _chunk, 0)

    def _drain(i, carry):
        pltpu.make_async_copy(ones_v, acc.at[didx.at[i]], dsem).wait()
        return carry

    lax.fori_loop(0, MCH, _drain, 0)
    plsc.subcore_barrier()
    pltpu.sync_copy(acc.at[pl.ds(base, RPT)], dbuf)

    def _b(i, carry):
        d16 = dbuf[pl.ds(i * 16, 16)]
        for k in range(16):
            vk = jnp.broadcast_to(d16[k], (16,))
            r = 8 * i + k // 2
            off = (k % 2) * 64
            for j in range(4):
                fbuf[r, pl.ds(off + j * 16, 16)] = vk
        return carry

    lax.fori_loop(0, RPT // 16, _b, 0)
    pltpu.sync_copy(fbuf, out_hbm.at[c, pl.ds(s * (RPT // 2), RPT // 2)])


_deg_call = pl.kernel(
    _deg_body,
    out_type=jax.ShapeDtypeStruct((NC, NFP, 2 * D_HID), jnp.float32),
    mesh=_mesh,
    scratch_types=[
        pltpu.VMEM((MCH, C), jnp.int32),
        pltpu.VMEM((C,), jnp.float32),
        pltpu.VMEM((RPT,), jnp.float32),
        pltpu.VMEM((RPT,), jnp.float32),
        pltpu.VMEM((RPT // 2, 2 * D_HID), jnp.float32),
        pltpu.VMEM_SHARED((NP,), jnp.float32),
        pltpu.SemaphoreType.DMA,
    ],
    compiler_params=pltpu.CompilerParams(use_tc_tiling_on_sc=False),
)


G = 6
NIT = (MCH + 2 * G - 1) // (2 * G)


def _prop_body(src_hbm, dst_hbm, z_hbm, out_hbm, sidx, didx, *rest):
    A = list(rest[:G])
    B = list(rest[G:2 * G])
    tmp, acc, gsa, ssa, gsb, ssb = rest[2 * G:]
    c = lax.axis_index("c")
    s = lax.axis_index("s")
    wid = s * NC + c
    pltpu.sync_copy(src_hbm.at[wid], sidx)
    pltpu.sync_copy(dst_hbm.at[wid], didx)

    def g_start(k, buf, sem):
        pltpu.async_copy(z_hbm.at[sidx.at[k]], buf, sem)

    def g_wait(k, buf, sem):
        pltpu.make_async_copy(z_hbm.at[sidx.at[k]], buf, sem).wait()

    def s_start(k, buf, sem):
        pltpu.async_copy(buf, acc.at[didx.at[k]], sem, add=True)

    def s_wait(k, buf, sem):
        pltpu.make_async_copy(buf, acc.at[didx.at[k]], sem).wait()

    for j in range(G):
        g_start(j, A[j], gsa.at[j])

    zero16 = jnp.zeros((16,), dtype=jnp.float32)

    def _z(i, carry):
        for j in range(D_HID // 16):
            tmp[i, pl.ds(j * 16, 16)] = zero16
        return carry

    lax.fori_loop(0, 128, _z, 0)
    base = s * RPT
    for j in range(RPT // 128):
        pltpu.sync_copy(tmp, acc.at[pl.ds(base + j * 128, 128)])
    plsc.subcore_barrier()

    def _iter(it, carry):
        i = it * 2 * G
        for j in range(G):
            def _a(j=j):
                g_wait(i + j, A[j], gsa.at[j])
                s_start(i + j, A[j], ssa.at[j])
            pl.when(i + j < MCH)(_a)
        for j in range(G):
            def _bd(j=j):
                s_wait(i - G + j, B[j], ssb.at[j])
            pl.when(it > 0)(_bd)
        for j in range(G):
            def _bg(j=j):
                g_start(i + G + j, B[j], gsb.at[j])
            pl.when(i + G + j < MCH)(_bg)
        for j in range(G):
            def _bw(j=j):
                g_wait(i + G + j, B[j], gsb.at[j])
            pl.when(i + G + j < MCH)(_bw)
        for j in range(G):
            def _ad(j=j):
                s_wait(i + j, A[j], ssa.at[j])
            pl.when(i + j < MCH)(_ad)
        for j in range(G):
            def _bs(j=j):
                s_start(i + G + j, B[j], ssb.at[j])
            pl.when(i + G + j < MCH)(_bs)
        for j in range(G):
            def _ag(j=j):
                g_start(i + 2 * G + j, A[j], gsa.at[j])
            pl.when(i + 2 * G + j < MCH)(_ag)
        return carry

    lax.fori_loop(0, NIT, _iter, 0)
    for j in range(G):
        k = (NIT - 1) * 2 * G + G + j
        if k < MCH:
            s_wait(k, B[j], ssb.at[j])
    plsc.subcore_barrier()
    for j in range(RPT // 128):
        sl = pl.ds(base + j * 128, 128)
        pltpu.sync_copy(acc.at[sl], out_hbm.at[c, sl])


_prop_call = pl.kernel(
    _prop_body,
    out_type=jax.ShapeDtypeStruct((NC, NP, D_HID), jnp.float32),
    mesh=_mesh,
    scratch_types=[
        pltpu.VMEM((MCH, C), jnp.int32),
        pltpu.VMEM((MCH, C), jnp.int32),
    ] + [pltpu.VMEM((C, D_HID), jnp.float32)] * (2 * G) + [
        pltpu.VMEM((128, D_HID), jnp.float32),
        pltpu.VMEM_SHARED((NP, D_HID), jnp.float32),
        pltpu.SemaphoreType.DMA((G,)),
        pltpu.SemaphoreType.DMA((G,)),
        pltpu.SemaphoreType.DMA((G,)),
        pltpu.SemaphoreType.DMA((G,)),
    ],
    compiler_params=pltpu.CompilerParams(use_tc_tiling_on_sc=False),
)


def _tcb_body(xf_ref, w2_ref, degf_ref, z1f_ref, dinvf_ref):
    deg = degf_ref[0][:NF] + degf_ref[1][:NF] + 1.0
    dinv = lax.rsqrt(deg)
    y = jnp.dot(xf_ref[...], w2_ref[...], preferred_element_type=jnp.float32)
    z1f_ref[...] = y * dinv
    dinvf_ref[...] = dinv


_tcb_call = pl.pallas_call(
    _tcb_body,
    out_shape=[
        jax.ShapeDtypeStruct((NF, 2 * D_HID), jnp.float32),
        jax.ShapeDtypeStruct((NF, 2 * D_HID), jnp.float32),
    ],
)


def _tcc_body(t1f_ref, z1f_ref, dinvf_ref, b1_ref, wc2_ref, z2f_ref):
    t = t1f_ref[0][:NF] + t1f_ref[1][:NF] + z1f_ref[...]
    dinv = dinvf_ref[...]
    h = jnp.maximum(dinv * t + b1_ref[...], 0.0)
    y2 = jnp.dot(h, wc2_ref[...], preferred_element_type=jnp.float32)
    z2f_ref[...] = y2 * dinv


_tcc_call = pl.pallas_call(
    _tcc_body,
    out_shape=jax.ShapeDtypeStruct((NF, 2 * D_HID), jnp.float32),
)


def _tcd_body(t2f_ref, z2f_ref, dinvf_ref, bc2_ref, of_ref):
    t = t2f_ref[0][:NF] + t2f_ref[1][:NF] + z2f_ref[...]
    of_ref[...] = dinvf_ref[...] * t + bc2_ref[...]


_tcd_call = pl.pallas_call(
    _tcd_body,
    out_shape=jax.ShapeDtypeStruct((NF, 2 * D_HID), jnp.float32),
)


def kernel(x, edge_index, W1, b1, Wmu, bmu, Wsig, bsig):
    f32 = jnp.float32
    src = edge_index[0].astype(jnp.int32).reshape(NW, MCH, C)
    dst = edge_index[1].astype(jnp.int32).reshape(NW, MCH, C)
    degf = _deg_call(dst)
    W2 = jnp.zeros((2 * D_IN, 2 * D_HID), f32)
    W2 = W2.at[:D_IN, :D_HID].set(W1).at[D_IN:, D_HID:].set(W1)
    wcat = jnp.concatenate([Wmu, Wsig], axis=1)
    wc2 = jnp.zeros((2 * D_HID, 2 * D_HID), f32)
    wc2 = wc2.at[:D_HID, :D_HID].set(wcat).at[D_HID:, D_HID:].set(wcat)
    b1_2 = jnp.concatenate([b1, b1])[None, :]
    bc2 = jnp.concatenate([bmu, bsig, bmu, bsig])[None, :]
    xf = x.reshape(NF, 2 * D_IN)
    z1f, dinvf = _tcb_call(xf, W2, degf)
    t1 = _prop_call(src, dst, z1f.reshape(N, D_HID))
    z2f = _tcc_call(t1.reshape(NC, NFP, 2 * D_HID), z1f, dinvf, b1_2, wc2)
    t2 = _prop_call(src, dst, z2f.reshape(N, D_HID))
    of = _tcd_call(t2.reshape(NC, NFP, 2 * D_HID), z2f, dinvf, bc2)
    o = of.reshape(N, D_HID)
    return o[:, :D_OUT], o[:, D_OUT:]
